# baseline (speedup 1.0000x reference)
import functools
import math

import jax
import jax.numpy as jnp
from jax.experimental import pallas as pl
from jax.experimental.pallas import tpu as pltpu

TEMP = 0.05
LN2 = math.log(2.0)
SCALE = 1.0 / (TEMP * LN2)
BK = 32768
NS = 2
NEG = -1e30


def _ce_block_kernel(x_ref, t_ref, f0_ref, f1_ref, out_ref,
                     m_ref, s_ref, tl_ref, *, nsteps, m_total, b):
    i = pl.program_id(0)

    @pl.when(i == 0)
    def _init():
        m_ref[...] = jnp.full_like(m_ref, NEG)
        s_ref[...] = jnp.zeros_like(s_ref)
        tl_ref[...] = jnp.zeros_like(tl_ref)

    x = x_ref[...]
    t = t_ref[...]

    def _update(zv):
        bm = jnp.max(zv, axis=1, keepdims=True)
        m_old = m_ref[...]
        m_new = jnp.maximum(m_old, bm)
        s_ref[...] = (s_ref[...] * jnp.exp2(m_old - m_new)
                      + jnp.sum(jnp.exp2(zv - m_new), axis=1, keepdims=True))
        m_ref[...] = m_new

    for k, f_ref in enumerate((f0_ref, f1_ref)):
        z = jax.lax.dot_general(
            x, f_ref[...], (((1,), (0,)), ((), ())),
            preferred_element_type=jnp.float32)
        cols = ((NS * i + k) * BK
                + jax.lax.broadcasted_iota(jnp.int32, (b, BK), 1))
        tl_ref[...] += jnp.sum(jnp.where(cols == t, z, 0.0), axis=1,
                               keepdims=True)

        @pl.when(i < nsteps - 1)
        def _body():
            _update(z)

        @pl.when(i == nsteps - 1)
        def _tail():
            _update(jnp.where(cols < m_total, z, NEG))

    @pl.when(i == nsteps - 1)
    def _fin():
        nll = LN2 * (m_ref[...] + jnp.log2(s_ref[...]) - tl_ref[...])
        out_ref[0, 0] = jnp.mean(nll)


@jax.jit
def kernel(inputs, targets, features):
    b, d = inputs.shape
    m, _ = features.shape
    nsteps = (m + NS * BK - 1) // (NS * BK)
    lastblk = (m - 1) // BK

    x = inputs * jnp.float32(SCALE)
    fT = features.T
    t2d = targets.astype(jnp.int32).reshape(b, 1)

    out = pl.pallas_call(
        functools.partial(_ce_block_kernel, nsteps=nsteps, m_total=m, b=b),
        grid=(nsteps,),
        in_specs=[
            pl.BlockSpec((b, d), lambda i: (0, 0)),
            pl.BlockSpec((b, 1), lambda i: (0, 0)),
            pl.BlockSpec((d, BK),
                         lambda i: (0, jnp.minimum(NS * i, lastblk))),
            pl.BlockSpec((d, BK),
                         lambda i: (0, jnp.minimum(NS * i + 1, lastblk))),
        ],
        out_specs=pl.BlockSpec(memory_space=pltpu.SMEM),
        out_shape=jax.ShapeDtypeStruct((1, 1), jnp.float32),
        scratch_shapes=[
            pltpu.VMEM((b, 1), jnp.float32),
            pltpu.VMEM((b, 1), jnp.float32),
            pltpu.VMEM((b, 1), jnp.float32),
        ],
        compiler_params=pltpu.CompilerParams(
            dimension_semantics=("arbitrary",)),
    )(x, t2d, fT, fT)
    return out[0, 0]


# speedup vs baseline: 1.0271x; 1.0271x over previous
import functools
import math

import jax
import jax.numpy as jnp
from jax.experimental import pallas as pl
from jax.experimental.pallas import tpu as pltpu

TEMP = 0.05
LN2 = math.log(2.0)
SCALE = 1.0 / (TEMP * LN2)
BK = 65536
NEG = -1e30


def _ce_block_kernel(x_ref, t_ref, feat_ref, out_ref, m_ref, s_ref, tl_ref,
                     *, nsteps, m_total, b):
    i = pl.program_id(0)

    @pl.when(i == 0)
    def _init():
        m_ref[...] = jnp.full_like(m_ref, NEG)
        s_ref[...] = jnp.zeros_like(s_ref)
        tl_ref[...] = jnp.zeros_like(tl_ref)

    x = x_ref[...]
    f = feat_ref[...]
    z = jax.lax.dot_general(
        x, f, (((1,), (0,)), ((), ())),
        preferred_element_type=jnp.float32)

    cols = i * BK + jax.lax.broadcasted_iota(jnp.int32, (b, BK), 1)
    t = t_ref[...]
    tl_ref[...] += jnp.sum(jnp.where(cols == t, z, 0.0), axis=1,
                           keepdims=True)

    def _update(zv):
        bm = jnp.max(zv, axis=1, keepdims=True)
        m_old = m_ref[...]
        m_new = jnp.maximum(m_old, bm)
        s_ref[...] = (s_ref[...] * jnp.exp2(m_old - m_new)
                      + jnp.sum(jnp.exp2(zv - m_new), axis=1, keepdims=True))
        m_ref[...] = m_new

    @pl.when(i < nsteps - 1)
    def _body():
        _update(z)

    @pl.when(i == nsteps - 1)
    def _fin():
        _update(jnp.where(cols < m_total, z, NEG))
        nll = LN2 * (m_ref[...] + jnp.log2(s_ref[...]) - tl_ref[...])
        out_ref[0, 0] = jnp.mean(nll)


@jax.jit
def kernel(inputs, targets, features):
    b, d = inputs.shape
    m, _ = features.shape
    nsteps = (m + BK - 1) // BK

    x = inputs * jnp.float32(SCALE)
    fT = features.T
    t2d = targets.astype(jnp.int32).reshape(b, 1)

    out = pl.pallas_call(
        functools.partial(_ce_block_kernel, nsteps=nsteps, m_total=m, b=b),
        grid=(nsteps,),
        in_specs=[
            pl.BlockSpec((b, d), lambda i: (0, 0)),
            pl.BlockSpec((b, 1), lambda i: (0, 0)),
            pl.BlockSpec((d, BK), lambda i: (0, i)),
        ],
        out_specs=pl.BlockSpec(memory_space=pltpu.SMEM),
        out_shape=jax.ShapeDtypeStruct((1, 1), jnp.float32),
        scratch_shapes=[
            pltpu.VMEM((b, 1), jnp.float32),
            pltpu.VMEM((b, 1), jnp.float32),
            pltpu.VMEM((b, 1), jnp.float32),
        ],
        compiler_params=pltpu.CompilerParams(
            dimension_semantics=("arbitrary",)),
    )(x, t2d, fT)
    return out[0, 0]
